# baseline (speedup 1.0000x reference)
import jax
import jax.numpy as jnp
from jax import lax
from jax.experimental import pallas as pl
from jax.experimental.pallas import tpu as pltpu
from jax.experimental.pallas import tpu_sc as plsc

DIM = 128
MAX_LEN = 2048
LANES = 16
WINDOW = 128
K = 10
N_OUTER = 20
KW = K * WINDOW
NBUF = 4
N_TILES = 32
LAG = 2


def _sc_gather(idx_flat, pe):
    B = idx_flat.shape[0]
    tile_rows = B // N_TILES
    assert tile_rows == N_OUTER * KW
    mesh = plsc.VectorSubcoreMesh(core_axis_name="core", subcore_axis_name="subcore")

    @pl.kernel(
        out_type=jax.ShapeDtypeStruct((B, DIM), pe.dtype),
        mesh=mesh,
        scratch_types=[
            pltpu.VMEM((2, KW), jnp.int32),
            pltpu.VMEM((NBUF, WINDOW), jnp.int32),
            pltpu.VMEM((NBUF, WINDOW, DIM), pe.dtype),
            pltpu.VMEM_SHARED((MAX_LEN, DIM), pe.dtype),
        ]
        + [pltpu.SemaphoreType.DMA] * (2 + 2 * NBUF),
    )
    def k(pe_hbm, i_hbm, o_hbm, ibuf, gidx, obuf, pe_sh, *sems):
        sem_i = sems[:2]
        sem_g = sems[2 : 2 + NBUF]
        sem_w = sems[2 + NBUF :]

        sid = lax.axis_index("subcore")
        chunk = MAX_LEN // 16
        pltpu.sync_copy(
            pe_hbm.at[pl.ds(sid * chunk, chunk)],
            pe_sh.at[pl.ds(sid * chunk, chunk)],
        )
        plsc.subcore_barrier()

        wid = sid * 2 + lax.axis_index("core")
        row_base = wid * tile_rows

        def wait_iblock(b):
            pltpu.make_async_copy(
                i_hbm.at[pl.ds(0, KW)], ibuf.at[b], sem_i[b]
            ).wait()

        def issue_iblock(b, blk):
            pltpu.async_copy(
                i_hbm.at[pl.ds(row_base + blk * KW, KW)], ibuf.at[b], sem_i[b]
            )

        def gather_src(j):
            return pe_hbm if j == K - 1 else pe_sh

        def wait_gather(s, j):
            pltpu.make_async_copy(
                gather_src(j).at[gidx.at[s]], obuf.at[s], sem_g[s]
            ).wait()

        def wait_write(s):
            pltpu.make_async_copy(
                obuf.at[s], o_hbm.at[pl.ds(0, WINDOW)], sem_w[s]
            ).wait()

        def issue_write(g, b, j):
            s = (K * b + j - LAG) % NBUF
            wait_gather(s, (j - LAG) % K)
            lin = (g + b) * K + (j - LAG)
            pltpu.async_copy(
                obuf.at[s],
                o_hbm.at[pl.ds(row_base + lin * WINDOW, WINDOW)],
                sem_w[s],
            )

        issue_iblock(0, 0)
        issue_iblock(1, 1)

        def phase(g, b):
            blk = g + b
            wait_iblock(b)
            row = ibuf.at[b]
            for j in range(K):
                s = (K * b + j) % NBUF
                if b == 1 or j >= NBUF:
                    wait_write(s)
                else:

                    @pl.when(g > 0)
                    def _():
                        wait_write(s)

                for c in range(0, WINDOW, LANES):
                    raw = row.at[pl.ds(j * WINDOW + c, LANES)][...]
                    gidx.at[s].at[pl.ds(c, LANES)][...] = jnp.minimum(
                        jnp.maximum(raw, 0), MAX_LEN - 1
                    )

                pltpu.async_copy(
                    gather_src(j).at[gidx.at[s]], obuf.at[s], sem_g[s]
                )

                if b == 1 or j >= LAG:
                    issue_write(g, b, j)
                else:

                    @pl.when(g > 0)
                    def _():
                        issue_write(g, b, j)

            @pl.when(blk + 2 < N_OUTER)
            def _():
                issue_iblock(b, blk + 2)

        @pl.loop(0, N_OUTER, step=2)
        def _(g):
            phase(g, 0)
            phase(g, 1)

        last = N_OUTER * K
        for lin in range(last - LAG, last):
            s = lin % NBUF
            wait_gather(s, lin % K)
            pltpu.async_copy(
                obuf.at[s],
                o_hbm.at[pl.ds(row_base + lin * WINDOW, WINDOW)],
                sem_w[s],
            )
        for s in range(NBUF):
            wait_write(s)

    return k(pe, idx_flat)


@jax.jit
def kernel(positions, pe):
    b0, b1 = positions.shape
    idx_flat = positions.reshape(b0 * b1)
    out = _sc_gather(idx_flat, pe)
    return out.reshape(b0, b1, DIM)


# speedup vs baseline: 1.0882x; 1.0882x over previous
import jax
import jax.numpy as jnp
from jax import lax
from jax.experimental import pallas as pl
from jax.experimental.pallas import tpu as pltpu
from jax.experimental.pallas import tpu_sc as plsc

DIM = 128
MAX_LEN = 2048
LANES = 16
WINDOW = 128
K = 10
N_OUTER = 20
KW = K * WINDOW
NBUF = 4
N_TILES = 32
LAG = 2


def _sc_gather(idx_flat, pe):
    B = idx_flat.shape[0]
    tile_rows = B // N_TILES
    assert tile_rows == N_OUTER * KW
    mesh = plsc.VectorSubcoreMesh(core_axis_name="core", subcore_axis_name="subcore")

    @pl.kernel(
        out_type=jax.ShapeDtypeStruct((B, DIM), pe.dtype),
        mesh=mesh,
        scratch_types=[
            pltpu.VMEM((2, KW), jnp.int32),
            pltpu.VMEM((NBUF, WINDOW), jnp.int32),
            pltpu.VMEM((NBUF, WINDOW, DIM), pe.dtype),
            pltpu.VMEM_SHARED((MAX_LEN, DIM), pe.dtype),
        ]
        + [pltpu.SemaphoreType.DMA] * (2 + 2 * NBUF),
    )
    def k(pe_hbm, i_hbm, o_hbm, ibuf, gidx, obuf, pe_sh, *sems):
        sem_i = sems[:2]
        sem_g = sems[2 : 2 + NBUF]
        sem_w = sems[2 + NBUF :]

        sid = lax.axis_index("subcore")
        chunk = MAX_LEN // 16
        pltpu.sync_copy(
            pe_hbm.at[pl.ds(sid * chunk, chunk)],
            pe_sh.at[pl.ds(sid * chunk, chunk)],
        )
        plsc.subcore_barrier()

        wid = sid * 2 + lax.axis_index("core")
        row_base = wid * tile_rows

        def wait_iblock(b):
            pltpu.make_async_copy(
                i_hbm.at[pl.ds(0, KW)], ibuf.at[b], sem_i[b]
            ).wait()

        def issue_iblock(b, blk):
            pltpu.async_copy(
                i_hbm.at[pl.ds(row_base + blk * KW, KW)], ibuf.at[b], sem_i[b]
            )

        def wait_gather(s):
            pltpu.make_async_copy(
                pe_sh.at[gidx.at[s]], obuf.at[s], sem_g[s]
            ).wait()

        def wait_write(s):
            pltpu.make_async_copy(
                obuf.at[s], o_hbm.at[pl.ds(0, WINDOW)], sem_w[s]
            ).wait()

        def issue_write(g, b, j):
            s = (K * b + j - LAG) % NBUF
            wait_gather(s)
            lin = (g + b) * K + (j - LAG)
            pltpu.async_copy(
                obuf.at[s],
                o_hbm.at[pl.ds(row_base + lin * WINDOW, WINDOW)],
                sem_w[s],
            )

        issue_iblock(0, 0)
        issue_iblock(1, 1)

        def phase(g, b):
            blk = g + b
            wait_iblock(b)
            row = ibuf.at[b]
            for j in range(K):
                s = (K * b + j) % NBUF
                if b == 1 or j >= NBUF:
                    wait_write(s)
                else:

                    @pl.when(g > 0)
                    def _():
                        wait_write(s)

                for c in range(0, WINDOW, LANES):
                    raw = row.at[pl.ds(j * WINDOW + c, LANES)][...]
                    gidx.at[s].at[pl.ds(c, LANES)][...] = jnp.minimum(
                        jnp.maximum(raw, 0), MAX_LEN - 1
                    )

                pltpu.async_copy(pe_sh.at[gidx.at[s]], obuf.at[s], sem_g[s])

                if b == 1 or j >= LAG:
                    issue_write(g, b, j)
                else:

                    @pl.when(g > 0)
                    def _():
                        issue_write(g, b, j)

            @pl.when(blk + 2 < N_OUTER)
            def _():
                issue_iblock(b, blk + 2)

        @pl.loop(0, N_OUTER, step=2)
        def _(g):
            phase(g, 0)
            phase(g, 1)

        last = N_OUTER * K
        for lin in range(last - LAG, last):
            s = lin % NBUF
            wait_gather(s)
            pltpu.async_copy(
                obuf.at[s],
                o_hbm.at[pl.ds(row_base + lin * WINDOW, WINDOW)],
                sem_w[s],
            )
        for s in range(NBUF):
            wait_write(s)

    return k(pe, idx_flat)


@jax.jit
def kernel(positions, pe):
    b0, b1 = positions.shape
    idx_flat = positions.reshape(b0 * b1)
    out = _sc_gather(idx_flat, pe)
    return out.reshape(b0, b1, DIM)
